# Initial kernel scaffold; baseline (speedup 1.0000x reference)
import jax
import jax.numpy as jnp
from jax.experimental import pallas as pl


def kernel(x, edge_index, W_in, b_in, Wc, W_out, b_out):
    raise NotImplementedError("write your pallas kernel here")


# speedup vs baseline: 3.6328x; 3.6328x over previous
import functools

import jax
import jax.numpy as jnp
from jax import lax
from jax.experimental import pallas as pl
from jax.experimental.pallas import tpu as pltpu
from jax.experimental.pallas import tpu_sc as plsc

N = 10000
E = 320000
H = 128
C = 40
L = 8
ALPHA = 0.1
THETA = 0.5

NP = 10240
BLK = 512
NW = 32
K = 128
NB = 80
EPAD = NW * NB * K
ROWS_PER_TILE = NP // 16


def _fill_const(ref, nrows, value):
    ncol = ref.shape[1] // 16
    def row(i, _):
        for kk in range(ncol):
            ref[i, pl.ds(kk * 16, 16)] = jnp.full((16,), value, jnp.float32)
        return 0
    lax.fori_loop(0, nrows, row, 0)


def _sc_degree(dst_p):
    mesh = plsc.VectorSubcoreMesh(core_axis_name="c", subcore_axis_name="s")

    @functools.partial(
        pl.kernel,
        out_type=jax.ShapeDtypeStruct((2, 16, ROWS_PER_TILE, 16), jnp.float32),
        mesh=mesh,
        scratch_types=[
            pltpu.VMEM((NB, K), jnp.int32),
            pltpu.VMEM((K, 16), jnp.float32),
            pltpu.VMEM((K, 16), jnp.float32),
            pltpu.VMEM((ROWS_PER_TILE // K, K), jnp.int32),
            pltpu.VMEM_SHARED((NP, 16), jnp.float32),
        ],
    )
    def deg_kernel(dst_hbm, out_hbm, dst_v, ones_v, zb, idz, deg_sh):
        c = lax.axis_index("c")
        s = lax.axis_index("s")
        w = c * 16 + s
        pltpu.sync_copy(dst_hbm.at[w], dst_v)
        _fill_const(ones_v, K, 1.0)
        _fill_const(zb, K, 0.0)
        base = s * ROWS_PER_TILE
        lanes = jnp.arange(16, dtype=jnp.int32)
        for b in range(ROWS_PER_TILE // K):
            for g2 in range(K // 16):
                idz[b, pl.ds(g2 * 16, 16)] = base + (b * K + g2 * 16) + lanes
        for b in range(ROWS_PER_TILE // K):
            pltpu.sync_copy(zb, deg_sh.at[idz.at[b]])
        plsc.subcore_barrier()

        def body(j, _):
            pltpu.sync_copy(ones_v, deg_sh.at[dst_v.at[j]], add=True)
            return 0
        lax.fori_loop(0, NB, body, 0)
        plsc.subcore_barrier()
        for b in range(ROWS_PER_TILE // K):
            pltpu.sync_copy(deg_sh.at[idz.at[b]], ones_v)
            pltpu.sync_copy(ones_v, out_hbm.at[c, s, pl.ds(b * K, K)])

    return deg_kernel(dst_p).reshape(2, NP, 16)


def _sc_propagate(src_p, dst_p, g):
    mesh = plsc.VectorSubcoreMesh(core_axis_name="c", subcore_axis_name="s")

    @functools.partial(
        pl.kernel,
        out_type=jax.ShapeDtypeStruct((2, 16, ROWS_PER_TILE, H), jnp.float32),
        mesh=mesh,
        scratch_types=[
            pltpu.VMEM((NB // 2, K), jnp.int32),
            pltpu.VMEM((NB // 2, K), jnp.int32),
            pltpu.VMEM((K, H), jnp.float32),
            pltpu.VMEM((K, H), jnp.float32),
            pltpu.VMEM((ROWS_PER_TILE // K, K), jnp.int32),
            pltpu.VMEM_SHARED((NP, H), jnp.float32),
            pltpu.SemaphoreType.DMA,
            pltpu.SemaphoreType.DMA,
        ],
    )
    def prop_kernel(src_hbm, dst_hbm, g_hbm, out_hbm,
                    src_v, dst_v, buf0, buf1, idz, agg_sh, sem0, sem1):
        c = lax.axis_index("c")
        s = lax.axis_index("s")
        w = c * 16 + s
        nh = NB // 2
        base = s * ROWS_PER_TILE
        lanes = jnp.arange(16, dtype=jnp.int32)
        for b in range(ROWS_PER_TILE // K):
            for g2 in range(K // 16):
                idz[b, pl.ds(g2 * 16, 16)] = base + (b * K + g2 * 16) + lanes
        _fill_const(buf0, K, 0.0)
        for b in range(ROWS_PER_TILE // K):
            pltpu.sync_copy(buf0, agg_sh.at[idz.at[b]])
        plsc.subcore_barrier()

        for ph in range(2):
            pltpu.sync_copy(src_hbm.at[w, pl.ds(ph * nh, nh)], src_v)
            pltpu.sync_copy(dst_hbm.at[w, pl.ds(ph * nh, nh)], dst_v)

            def body(j, _):
                pltpu.async_copy(g_hbm.at[src_v.at[j]], buf0, sem0).wait()
                pltpu.sync_copy(buf0, agg_sh.at[dst_v.at[j]], add=True)
                return 0
            lax.fori_loop(0, nh, body, 0)
        plsc.subcore_barrier()
        for b in range(ROWS_PER_TILE // K):
            pltpu.sync_copy(agg_sh.at[idz.at[b]], buf0)
            pltpu.sync_copy(buf0, out_hbm.at[c, s, pl.ds(b * K, K)])

    return prop_kernel(src_p, dst_p, g).reshape(2, NP, H)


def _tc_input(x_p, w_in, b_in, degp):
    def body(x_ref, w_ref, b_ref, d_ref, h_ref, g_ref, dinv_ref):
        deg = d_ref[0, :, 0:1] + d_ref[1, :, 0:1] + 1.0
        dinv = lax.rsqrt(deg)
        h = jnp.dot(x_ref[...], w_ref[...],
                    preferred_element_type=jnp.float32) + b_ref[...]
        h_ref[...] = h
        g_ref[...] = dinv * h
        dinv_ref[...] = dinv

    grid = (NP // BLK,)
    return pl.pallas_call(
        body,
        grid=grid,
        in_specs=[
            pl.BlockSpec((BLK, H), lambda i: (i, 0)),
            pl.BlockSpec((H, H), lambda i: (0, 0)),
            pl.BlockSpec((1, H), lambda i: (0, 0)),
            pl.BlockSpec((2, BLK, 16), lambda i: (0, i, 0)),
        ],
        out_specs=[
            pl.BlockSpec((BLK, H), lambda i: (i, 0)),
            pl.BlockSpec((BLK, H), lambda i: (i, 0)),
            pl.BlockSpec((BLK, 1), lambda i: (i, 0)),
        ],
        out_shape=[
            jax.ShapeDtypeStruct((NP, H), jnp.float32),
            jax.ShapeDtypeStruct((NP, H), jnp.float32),
            jax.ShapeDtypeStruct((NP, 1), jnp.float32),
        ],
    )(x_p, w_in, b_in.reshape(1, H), degp)


def _tc_layer(h, g, x0, parts, dinv, wc, beta):
    one_m_beta = 1.0 - beta

    def body(h_ref, g_ref, x0_ref, p_ref, dinv_ref, wc_ref, hn_ref, gn_ref):
        dinv = dinv_ref[...]
        agg = dinv * (p_ref[0] + p_ref[1] + g_ref[...])
        xi = (1.0 - ALPHA) * agg + ALPHA * x0_ref[...]
        t = jnp.dot(xi, wc_ref[...], preferred_element_type=jnp.float32)
        z = one_m_beta * xi + beta * t
        hn = jnp.where(z > 0.0, z, jnp.exp(jnp.minimum(z, 0.0)) - 1.0)
        hn_ref[...] = hn
        gn_ref[...] = dinv * hn

    grid = (NP // BLK,)
    return pl.pallas_call(
        body,
        grid=grid,
        in_specs=[
            pl.BlockSpec((BLK, H), lambda i: (i, 0)),
            pl.BlockSpec((BLK, H), lambda i: (i, 0)),
            pl.BlockSpec((BLK, H), lambda i: (i, 0)),
            pl.BlockSpec((2, BLK, H), lambda i: (0, i, 0)),
            pl.BlockSpec((BLK, 1), lambda i: (i, 0)),
            pl.BlockSpec((H, H), lambda i: (0, 0)),
        ],
        out_specs=[
            pl.BlockSpec((BLK, H), lambda i: (i, 0)),
            pl.BlockSpec((BLK, H), lambda i: (i, 0)),
        ],
        out_shape=[
            jax.ShapeDtypeStruct((NP, H), jnp.float32),
            jax.ShapeDtypeStruct((NP, H), jnp.float32),
        ],
    )(h, g, x0, parts, dinv, wc)


def _tc_output(h, w_out_p, b_out_p):
    def body(h_ref, w_ref, b_ref, o_ref):
        o = jnp.dot(h_ref[...], w_ref[...],
                    preferred_element_type=jnp.float32) + b_ref[...]
        col = lax.broadcasted_iota(jnp.int32, (BLK, H), 1)
        z = jnp.where(col < C, o, -jnp.inf)
        m = jnp.max(z, axis=1, keepdims=True)
        lse = jnp.log(jnp.sum(jnp.exp(z - m), axis=1, keepdims=True)) + m
        o_ref[...] = z - lse

    grid = (NP // BLK,)
    return pl.pallas_call(
        body,
        grid=grid,
        in_specs=[
            pl.BlockSpec((BLK, H), lambda i: (i, 0)),
            pl.BlockSpec((H, H), lambda i: (0, 0)),
            pl.BlockSpec((1, H), lambda i: (0, 0)),
        ],
        out_specs=pl.BlockSpec((BLK, H), lambda i: (i, 0)),
        out_shape=jax.ShapeDtypeStruct((NP, H), jnp.float32),
    )(h, w_out_p, b_out_p)


def kernel(x, edge_index, W_in, b_in, Wc, W_out, b_out):
    import numpy as np

    src = edge_index[0].astype(jnp.int32)
    dst = edge_index[1].astype(jnp.int32)
    pad = jnp.full((EPAD - E,), N, jnp.int32)
    src_p = jnp.concatenate([src, pad]).reshape(NW, NB, K)
    dst_p = jnp.concatenate([dst, pad]).reshape(NW, NB, K)
    x_p = jnp.pad(x, ((0, NP - N), (0, 0)))
    w_out_p = jnp.pad(W_out, ((0, 0), (0, H - C)))
    b_out_p = jnp.pad(b_out, (0, H - C)).reshape(1, H)

    degp = _sc_degree(dst_p)
    h, g, dinv = _tc_input(x_p, W_in, b_in, degp)
    x0 = h
    for l in range(L):
        parts = _sc_propagate(src_p, dst_p, g)
        beta = float(np.log(THETA / (l + 1) + 1.0))
        h, g = _tc_layer(h, g, x0, parts, dinv, Wc[l], beta)
    out = _tc_output(h, w_out_p, b_out_p)
    return out[:N, :C]


# speedup vs baseline: 4.0788x; 1.1228x over previous
import functools

import jax
import jax.numpy as jnp
from jax import lax
from jax.experimental import pallas as pl
from jax.experimental.pallas import tpu as pltpu
from jax.experimental.pallas import tpu_sc as plsc

N = 10000
E = 320000
H = 128
C = 40
L = 8
ALPHA = 0.1
THETA = 0.5

NP = 10240
BLK = 512
NW = 32
K = 128
NB = 80
EPAD = NW * NB * K
ROWS_PER_TILE = NP // 16


def _fill_const(ref, nrows, value):
    ncol = ref.shape[1] // 16
    def row(i, _):
        for kk in range(ncol):
            ref[i, pl.ds(kk * 16, 16)] = jnp.full((16,), value, jnp.float32)
        return 0
    lax.fori_loop(0, nrows, row, 0)


def _sc_degree(dst_p):
    mesh = plsc.VectorSubcoreMesh(core_axis_name="c", subcore_axis_name="s")

    @functools.partial(
        pl.kernel,
        out_type=jax.ShapeDtypeStruct((2, 16, ROWS_PER_TILE, 16), jnp.float32),
        mesh=mesh,
        scratch_types=[
            pltpu.VMEM((NB, K), jnp.int32),
            pltpu.VMEM((K, 16), jnp.float32),
            pltpu.VMEM((K, 16), jnp.float32),
            pltpu.VMEM((ROWS_PER_TILE // K, K), jnp.int32),
            pltpu.VMEM_SHARED((NP, 16), jnp.float32),
        ],
    )
    def deg_kernel(dst_hbm, out_hbm, dst_v, ones_v, zb, idz, deg_sh):
        c = lax.axis_index("c")
        s = lax.axis_index("s")
        w = c * 16 + s
        pltpu.sync_copy(dst_hbm.at[w], dst_v)
        _fill_const(ones_v, K, 1.0)
        _fill_const(zb, K, 0.0)
        base = s * ROWS_PER_TILE
        lanes = jnp.arange(16, dtype=jnp.int32)
        for b in range(ROWS_PER_TILE // K):
            for g2 in range(K // 16):
                idz[b, pl.ds(g2 * 16, 16)] = base + (b * K + g2 * 16) + lanes
        for b in range(ROWS_PER_TILE // K):
            pltpu.sync_copy(zb, deg_sh.at[idz.at[b]])
        plsc.subcore_barrier()

        def body(j, _):
            pltpu.sync_copy(ones_v, deg_sh.at[dst_v.at[j]], add=True)
            return 0
        lax.fori_loop(0, NB, body, 0)
        plsc.subcore_barrier()
        for b in range(ROWS_PER_TILE // K):
            pltpu.sync_copy(deg_sh.at[idz.at[b]], ones_v)
            pltpu.sync_copy(ones_v, out_hbm.at[c, s, pl.ds(b * K, K)])

    return deg_kernel(dst_p).reshape(2, NP, 16)


def _sc_propagate(src_p, dst_p, g):
    mesh = plsc.VectorSubcoreMesh(core_axis_name="c", subcore_axis_name="s")

    @functools.partial(
        pl.kernel,
        out_type=jax.ShapeDtypeStruct((2, 16, ROWS_PER_TILE, H), jnp.float32),
        mesh=mesh,
        scratch_types=[
            pltpu.VMEM((NB // 2, K), jnp.int32),
            pltpu.VMEM((NB // 2, K), jnp.int32),
            pltpu.VMEM((K, H), jnp.float32),
            pltpu.VMEM((K, H), jnp.float32),
            pltpu.VMEM((ROWS_PER_TILE // K, K), jnp.int32),
            pltpu.VMEM_SHARED((NP, H), jnp.float32),
            pltpu.SemaphoreType.DMA,
            pltpu.SemaphoreType.DMA,
            pltpu.SemaphoreType.DMA,
            pltpu.SemaphoreType.DMA,
        ],
    )
    def prop_kernel(src_hbm, dst_hbm, g_hbm, out_hbm,
                    src_v, dst_v, buf0, buf1, idz, agg_sh, gs0, gs1, ss0, ss1):
        c = lax.axis_index("c")
        s = lax.axis_index("s")
        w = c * 16 + s
        nh = NB // 2
        base = s * ROWS_PER_TILE
        lanes = jnp.arange(16, dtype=jnp.int32)
        for b in range(ROWS_PER_TILE // K):
            for g2 in range(K // 16):
                idz[b, pl.ds(g2 * 16, 16)] = base + (b * K + g2 * 16) + lanes
        _fill_const(buf0, K, 0.0)
        for b in range(ROWS_PER_TILE // K):
            pltpu.sync_copy(buf0, agg_sh.at[idz.at[b]])
        plsc.subcore_barrier()

        def start_g(j, buf, sem):
            pltpu.async_copy(g_hbm.at[src_v.at[j]], buf, sem)

        def wait_g(j, buf, sem):
            pltpu.make_async_copy(g_hbm.at[src_v.at[j]], buf, sem).wait()

        def start_s(j, buf, sem):
            pltpu.async_copy(buf, agg_sh.at[dst_v.at[j]], sem, add=True)

        def wait_s(j, buf, sem):
            pltpu.make_async_copy(buf, agg_sh.at[dst_v.at[j]], sem).wait()

        for ph in range(2):
            pltpu.sync_copy(src_hbm.at[w, pl.ds(ph * nh, nh)], src_v)
            pltpu.sync_copy(dst_hbm.at[w, pl.ds(ph * nh, nh)], dst_v)
            start_g(0, buf0, gs0)

            def body(i, _):
                j = 2 * i
                start_g(j + 1, buf1, gs1)
                wait_g(j, buf0, gs0)
                pltpu.sync_copy(buf0, agg_sh.at[dst_v.at[j]], add=True)
                start_g(j + 2, buf0, gs0)
                wait_g(j + 1, buf1, gs1)
                pltpu.sync_copy(buf1, agg_sh.at[dst_v.at[j + 1]], add=True)
                return 0
            lax.fori_loop(0, nh // 2 - 1, body, 0)
            j = nh - 2
            start_g(j + 1, buf1, gs1)
            wait_g(j, buf0, gs0)
            pltpu.sync_copy(buf0, agg_sh.at[dst_v.at[j]], add=True)
            wait_g(j + 1, buf1, gs1)
            pltpu.sync_copy(buf1, agg_sh.at[dst_v.at[j + 1]], add=True)
        plsc.subcore_barrier()
        for b in range(ROWS_PER_TILE // K):
            pltpu.sync_copy(agg_sh.at[idz.at[b]], buf0)
            pltpu.sync_copy(buf0, out_hbm.at[c, s, pl.ds(b * K, K)])

    return prop_kernel(src_p, dst_p, g).reshape(2, NP, H)


def _tc_input(x_p, w_in, b_in, degp):
    def body(x_ref, w_ref, b_ref, d_ref, h_ref, g_ref, dinv_ref):
        deg = d_ref[0, :, 0:1] + d_ref[1, :, 0:1] + 1.0
        dinv = lax.rsqrt(deg)
        h = jnp.dot(x_ref[...], w_ref[...],
                    preferred_element_type=jnp.float32) + b_ref[...]
        h_ref[...] = h
        g_ref[...] = dinv * h
        dinv_ref[...] = dinv

    grid = (NP // BLK,)
    return pl.pallas_call(
        body,
        grid=grid,
        in_specs=[
            pl.BlockSpec((BLK, H), lambda i: (i, 0)),
            pl.BlockSpec((H, H), lambda i: (0, 0)),
            pl.BlockSpec((1, H), lambda i: (0, 0)),
            pl.BlockSpec((2, BLK, 16), lambda i: (0, i, 0)),
        ],
        out_specs=[
            pl.BlockSpec((BLK, H), lambda i: (i, 0)),
            pl.BlockSpec((BLK, H), lambda i: (i, 0)),
            pl.BlockSpec((BLK, 1), lambda i: (i, 0)),
        ],
        out_shape=[
            jax.ShapeDtypeStruct((NP, H), jnp.float32),
            jax.ShapeDtypeStruct((NP, H), jnp.float32),
            jax.ShapeDtypeStruct((NP, 1), jnp.float32),
        ],
    )(x_p, w_in, b_in.reshape(1, H), degp)


def _tc_layer(h, g, x0, parts, dinv, wc, beta):
    one_m_beta = 1.0 - beta

    def body(h_ref, g_ref, x0_ref, p_ref, dinv_ref, wc_ref, hn_ref, gn_ref):
        dinv = dinv_ref[...]
        agg = dinv * (p_ref[0] + p_ref[1] + g_ref[...])
        xi = (1.0 - ALPHA) * agg + ALPHA * x0_ref[...]
        t = jnp.dot(xi, wc_ref[...], preferred_element_type=jnp.float32)
        z = one_m_beta * xi + beta * t
        hn = jnp.where(z > 0.0, z, jnp.exp(jnp.minimum(z, 0.0)) - 1.0)
        hn_ref[...] = hn
        gn_ref[...] = dinv * hn

    grid = (NP // BLK,)
    return pl.pallas_call(
        body,
        grid=grid,
        in_specs=[
            pl.BlockSpec((BLK, H), lambda i: (i, 0)),
            pl.BlockSpec((BLK, H), lambda i: (i, 0)),
            pl.BlockSpec((BLK, H), lambda i: (i, 0)),
            pl.BlockSpec((2, BLK, H), lambda i: (0, i, 0)),
            pl.BlockSpec((BLK, 1), lambda i: (i, 0)),
            pl.BlockSpec((H, H), lambda i: (0, 0)),
        ],
        out_specs=[
            pl.BlockSpec((BLK, H), lambda i: (i, 0)),
            pl.BlockSpec((BLK, H), lambda i: (i, 0)),
        ],
        out_shape=[
            jax.ShapeDtypeStruct((NP, H), jnp.float32),
            jax.ShapeDtypeStruct((NP, H), jnp.float32),
        ],
    )(h, g, x0, parts, dinv, wc)


def _tc_output(h, w_out_p, b_out_p):
    def body(h_ref, w_ref, b_ref, o_ref):
        o = jnp.dot(h_ref[...], w_ref[...],
                    preferred_element_type=jnp.float32) + b_ref[...]
        col = lax.broadcasted_iota(jnp.int32, (BLK, H), 1)
        z = jnp.where(col < C, o, -jnp.inf)
        m = jnp.max(z, axis=1, keepdims=True)
        lse = jnp.log(jnp.sum(jnp.exp(z - m), axis=1, keepdims=True)) + m
        o_ref[...] = z - lse

    grid = (NP // BLK,)
    return pl.pallas_call(
        body,
        grid=grid,
        in_specs=[
            pl.BlockSpec((BLK, H), lambda i: (i, 0)),
            pl.BlockSpec((H, H), lambda i: (0, 0)),
            pl.BlockSpec((1, H), lambda i: (0, 0)),
        ],
        out_specs=pl.BlockSpec((BLK, H), lambda i: (i, 0)),
        out_shape=jax.ShapeDtypeStruct((NP, H), jnp.float32),
    )(h, w_out_p, b_out_p)


def kernel(x, edge_index, W_in, b_in, Wc, W_out, b_out):
    import numpy as np

    src = edge_index[0].astype(jnp.int32)
    dst = edge_index[1].astype(jnp.int32)
    pad = jnp.full((EPAD - E,), N, jnp.int32)
    src_p = jnp.concatenate([src, pad]).reshape(NW, NB, K)
    dst_p = jnp.concatenate([dst, pad]).reshape(NW, NB, K)
    x_p = jnp.pad(x, ((0, NP - N), (0, 0)))
    w_out_p = jnp.pad(W_out, ((0, 0), (0, H - C)))
    b_out_p = jnp.pad(b_out, (0, H - C)).reshape(1, H)

    degp = _sc_degree(dst_p)
    h, g, dinv = _tc_input(x_p, W_in, b_in, degp)
    x0 = h
    for l in range(L):
        parts = _sc_propagate(src_p, dst_p, g)
        beta = float(np.log(THETA / (l + 1) + 1.0))
        h, g = _tc_layer(h, g, x0, parts, dinv, Wc[l], beta)
    out = _tc_output(h, w_out_p, b_out_p)
    return out[:N, :C]
